# baseline (speedup 1.0000x reference)
import functools

import jax
import jax.numpy as jnp
from jax import lax
from jax.experimental import pallas as pl
from jax.experimental.pallas import tpu as pltpu
from jax.experimental.pallas import tpu_sc as plsc

_L = 16


@functools.lru_cache(maxsize=None)
def _build_sc(n_rows, datasize, class_num):
    info = plsc.get_sparse_core_info()
    nw = info.num_cores * info.num_subcores
    b_per_w = n_rows // nw

    win = _L
    nwin = b_per_w // win
    nbuf = 4

    mesh = plsc.VectorSubcoreMesh(core_axis_name="c", subcore_axis_name="s")

    @functools.partial(
        pl.kernel,
        mesh=mesh,
        out_type=jax.ShapeDtypeStruct((n_rows, class_num), jnp.float32),
        scratch_types=[
            pltpu.VMEM((b_per_w,), jnp.int32),
            pltpu.VMEM((nbuf, win, class_num), jnp.float32),
            pltpu.SemaphoreType.DMA((nbuf,)),
            pltpu.SemaphoreType.DMA((nbuf,)),
        ],
        cost_estimate=pl.CostEstimate(
            flops=0,
            bytes_accessed=2 * n_rows * class_num * 4,
            transcendentals=0,
        ),
        compiler_params=pltpu.CompilerParams(has_side_effects=False),
    )
    def gather_kernel(idx_hbm, table_hbm, out_hbm, idx_v, obuf, gsem, wsem):
        wid = lax.axis_index("s") * info.num_cores + lax.axis_index("c")
        base = wid * b_per_w
        pltpu.sync_copy(idx_hbm.at[pl.ds(base, b_per_w)], idx_v)

        def fire(j):
            b = j % nbuf
            v = idx_v[pl.ds(j * win, _L)]
            for l in range(_L):
                i = lax.squeeze(lax.slice(v, [l], [l + 1]), [0])
                pltpu.async_copy(table_hbm.at[i], obuf.at[b, l], gsem.at[b])

        writes = [None] * nwin
        fire(0)
        for j in range(nwin):
            b = j % nbuf
            if j + 1 < nwin:
                if j + 1 >= nbuf:
                    writes[j + 1 - nbuf].wait()
                fire(j + 1)
            pltpu.make_async_copy(
                table_hbm.at[pl.ds(0, win)], obuf.at[b], gsem.at[b]
            ).wait()
            writes[j] = pltpu.async_copy(
                obuf.at[b],
                out_hbm.at[pl.ds(base + j * win, win)],
                wsem.at[b],
            )
        for j in range(max(nwin - nbuf, 0), nwin):
            writes[j].wait()

    return gather_kernel


@functools.lru_cache(maxsize=None)
def _build_tc(n_rows, datasize, class_num):
    chunk = n_rows
    ring = 16

    def body(idx_ref, table_ref, out_ref, sem):
        def step(j, carry):
            i = idx_ref[j]
            pltpu.async_copy(
                table_ref.at[pl.ds(i, 1)], out_ref.at[pl.ds(j, 1)], sem)

            @pl.when(j >= ring)
            def _drain():
                pltpu.make_async_copy(
                    table_ref.at[pl.ds(0, 1)], out_ref.at[pl.ds(0, 1)], sem
                ).wait()

            return carry

        lax.fori_loop(0, chunk, step, 0)
        for _ in range(ring):
            pltpu.make_async_copy(
                table_ref.at[pl.ds(0, 1)], out_ref.at[pl.ds(0, 1)], sem
            ).wait()

    return pl.pallas_call(
        body,
        grid=(n_rows // chunk,),
        in_specs=[
            pl.BlockSpec((chunk,), lambda c: (c,), memory_space=pltpu.SMEM),
            pl.BlockSpec(memory_space=pl.ANY),
        ],
        out_specs=pl.BlockSpec((chunk, class_num), lambda c: (c, 0)),
        out_shape=jax.ShapeDtypeStruct((n_rows, class_num), jnp.float32),
        scratch_shapes=[pltpu.SemaphoreType.DMA],
        cost_estimate=pl.CostEstimate(
            flops=0,
            bytes_accessed=2 * n_rows * class_num * 4,
            transcendentals=0,
        ),
        compiler_params=pltpu.CompilerParams(has_side_effects=False),
    )


_N_TC = 4608


def kernel(index, pseudo_label):
    batch = index.shape[0]
    datasize, class_num = pseudo_label.shape
    n_sc = batch - _N_TC
    out_tc = _build_tc(_N_TC, datasize, class_num)(
        index[n_sc:], pseudo_label)
    out_sc = _build_sc(n_sc, datasize, class_num)(
        index[:n_sc], pseudo_label)
    return jnp.concatenate([out_sc, out_tc], axis=0)


# speedup vs baseline: 1.4396x; 1.4396x over previous
import functools

import jax
import jax.numpy as jnp
from jax import lax
from jax.experimental import pallas as pl
from jax.experimental.pallas import tpu as pltpu
from jax.experimental.pallas import tpu_sc as plsc

_L = 16


@functools.lru_cache(maxsize=None)
def _build(batch, datasize, class_num):
    info = plsc.get_sparse_core_info()
    nw = info.num_cores * info.num_subcores
    b_per_w = batch // nw

    win = _L
    nwin = b_per_w // win
    nbuf = 4

    mesh = plsc.VectorSubcoreMesh(core_axis_name="c", subcore_axis_name="s")

    @functools.partial(
        pl.kernel,
        mesh=mesh,
        out_type=jax.ShapeDtypeStruct((batch, class_num), jnp.float32),
        scratch_types=[
            pltpu.VMEM((b_per_w,), jnp.int32),
            pltpu.VMEM((nbuf, win, class_num), jnp.float32),
            pltpu.SemaphoreType.DMA((nbuf,)),
            pltpu.SemaphoreType.DMA((nbuf,)),
        ],
    )
    def gather_kernel(idx_hbm, table_hbm, out_hbm, idx_v, obuf, gsem, wsem):
        wid = lax.axis_index("s") * info.num_cores + lax.axis_index("c")
        base = wid * b_per_w
        pltpu.sync_copy(idx_hbm.at[pl.ds(base, b_per_w)], idx_v)

        def fire(j):
            b = j % nbuf
            v = idx_v[pl.ds(j * win, _L)]
            for l in range(_L):
                i = lax.squeeze(lax.slice(v, [l], [l + 1]), [0])
                pltpu.async_copy(table_hbm.at[i], obuf.at[b, l], gsem.at[b])

        writes = [None] * nwin
        fire(0)
        for j in range(nwin):
            b = j % nbuf
            if j + 1 < nwin:
                if j + 1 >= nbuf:
                    writes[j + 1 - nbuf].wait()
                fire(j + 1)
            pltpu.make_async_copy(
                table_hbm.at[pl.ds(0, win)], obuf.at[b], gsem.at[b]
            ).wait()
            writes[j] = pltpu.async_copy(
                obuf.at[b],
                out_hbm.at[pl.ds(base + j * win, win)],
                wsem.at[b],
            )
        for j in range(max(nwin - nbuf, 0), nwin):
            writes[j].wait()

    return gather_kernel


def kernel(index, pseudo_label):
    batch = index.shape[0]
    datasize, class_num = pseudo_label.shape
    return _build(batch, datasize, class_num)(index, pseudo_label)
